# baseline (speedup 1.0000x reference)
import functools

import jax
import jax.numpy as jnp
from jax import lax
from jax.experimental import pallas as pl
from jax.experimental.pallas import tpu as pltpu
from jax.experimental.pallas import tpu_sc as plsc

D_MODEL = 128
NUM_CORES = 2
NUM_SUBCORES = 16
NUM_WORKERS = NUM_CORES * NUM_SUBCORES

CHUNK = 256
NBUF = 2


@functools.lru_cache(maxsize=None)
def _make_gather(batch: int, seq: int):
    B = batch * seq
    assert B % NUM_WORKERS == 0
    b_per_w = B // NUM_WORKERS
    assert seq % b_per_w == 0
    w_per_row = seq // b_per_w
    assert b_per_w % (CHUNK * NBUF) == 0
    n_steps = b_per_w // (CHUNK * NBUF)
    mesh = plsc.VectorSubcoreMesh(core_axis_name="c", subcore_axis_name="s")

    @functools.partial(
        pl.kernel,
        mesh=mesh,
        out_type=jax.ShapeDtypeStruct((B, D_MODEL), jnp.float32),
        scratch_types=[
            pltpu.VMEM((b_per_w,), jnp.int32),
            pltpu.VMEM((NBUF, CHUNK, D_MODEL), jnp.float32),
        ]
        + [pltpu.SemaphoreType.DMA] * (2 * NBUF),
    )
    def grab(idx_hbm, table_hbm, out_hbm, idx_v, rows_v, *sems):
        gsem, wsem = sems[:NBUF], sems[NBUF:]
        wid = lax.axis_index("s") * NUM_CORES + lax.axis_index("c")
        base = wid * b_per_w
        row = wid // w_per_row
        col = (wid % w_per_row) * b_per_w
        pltpu.sync_copy(idx_hbm.at[row, pl.ds(col, b_per_w)], idx_v)

        def step(g, carry):
            cbase = g * (CHUNK * NBUF)
            gathers = []
            for b in range(NBUF):
                @pl.when(g > 0)
                def _():
                    pltpu.make_async_copy(
                        rows_v.at[b],
                        out_hbm.at[pl.ds(base, CHUNK)],
                        wsem[b],
                    ).wait()

                gathers.append(
                    pltpu.async_copy(
                        table_hbm.at[idx_v.at[pl.ds(cbase + b * CHUNK, CHUNK)]],
                        rows_v.at[b],
                        gsem[b],
                    )
                )
            for b in range(NBUF):
                gathers[b].wait()
                pltpu.async_copy(
                    rows_v.at[b],
                    out_hbm.at[pl.ds(base + cbase + b * CHUNK, CHUNK)],
                    wsem[b],
                )
            return carry

        lax.fori_loop(0, n_steps, step, 0)
        for b in range(NBUF):
            pltpu.make_async_copy(
                rows_v.at[b], out_hbm.at[pl.ds(base, CHUNK)], wsem[b]
            ).wait()

    return grab


def kernel(x, table):
    batch, seq = x.shape
    out = _make_gather(batch, seq)(x.astype(jnp.int32), table)
    return out.reshape(batch, seq, D_MODEL)


# speedup vs baseline: 1.0094x; 1.0094x over previous
import functools

import jax
import jax.numpy as jnp
from jax import lax
from jax.experimental import pallas as pl
from jax.experimental.pallas import tpu as pltpu
from jax.experimental.pallas import tpu_sc as plsc

D_MODEL = 128
NUM_CORES = 2
NUM_SUBCORES = 16
NUM_WORKERS = NUM_CORES * NUM_SUBCORES

CHUNK = 256
NBUF = 3
DEPTH = 2


def _chunk_sizes(b_per_w: int):
    if b_per_w % CHUNK == 0 and b_per_w >= 2 * CHUNK:
        half = CHUNK // 2
        return [half] + [CHUNK] * (b_per_w // CHUNK - 1) + [half]
    assert b_per_w % CHUNK == 0
    return [CHUNK] * (b_per_w // CHUNK)


@functools.lru_cache(maxsize=None)
def _make_gather(batch: int, seq: int):
    B = batch * seq
    assert B % NUM_WORKERS == 0
    b_per_w = B // NUM_WORKERS
    assert seq % b_per_w == 0
    w_per_row = seq // b_per_w
    sizes = _chunk_sizes(b_per_w)
    offs = [sum(sizes[:i]) for i in range(len(sizes))]
    n_chunks = len(sizes)
    mesh = plsc.VectorSubcoreMesh(core_axis_name="c", subcore_axis_name="s")

    @functools.partial(
        pl.kernel,
        mesh=mesh,
        out_type=jax.ShapeDtypeStruct((B, D_MODEL), jnp.float32),
        scratch_types=[
            pltpu.VMEM((b_per_w,), jnp.int32),
            pltpu.VMEM((NBUF, CHUNK, D_MODEL), jnp.float32),
        ]
        + [pltpu.SemaphoreType.DMA] * (2 * NBUF + n_chunks),
    )
    def grab(idx_hbm, table_hbm, out_hbm, idx_v, rows_v, *sems):
        gsem, wsem = sems[:NBUF], sems[NBUF : 2 * NBUF]
        isem = sems[2 * NBUF :]
        wid = lax.axis_index("s") * NUM_CORES + lax.axis_index("c")
        base = wid * b_per_w
        row = wid // w_per_row
        col = (wid % w_per_row) * b_per_w

        idx_loads = [
            pltpu.async_copy(
                idx_hbm.at[row, pl.ds(col + offs[c], sizes[c])],
                idx_v.at[pl.ds(offs[c], sizes[c])],
                isem[c],
            )
            for c in range(n_chunks)
        ]

        gathers = [None] * n_chunks
        writes = [None] * n_chunks

        def start_write(d):
            gathers[d].wait()
            writes[d] = pltpu.async_copy(
                rows_v.at[d % NBUF, pl.ds(0, sizes[d])],
                out_hbm.at[pl.ds(base + offs[d], sizes[d])],
                wsem[d % NBUF],
            )

        for c in range(n_chunks):
            b = c % NBUF
            if c >= NBUF:
                writes[c - NBUF].wait()
            idx_loads[c].wait()
            gathers[c] = pltpu.async_copy(
                table_hbm.at[idx_v.at[pl.ds(offs[c], sizes[c])]],
                rows_v.at[b, pl.ds(0, sizes[c])],
                gsem[b],
            )
            if c - (DEPTH - 1) >= 0:
                start_write(c - (DEPTH - 1))
        for d in range(n_chunks - (DEPTH - 1), n_chunks):
            start_write(d)
        for d in range(max(0, n_chunks - NBUF), n_chunks):
            writes[d].wait()

    return grab


def kernel(x, table):
    batch, seq = x.shape
    out = _make_gather(batch, seq)(x.astype(jnp.int32), table)
    return out.reshape(batch, seq, D_MODEL)


# speedup vs baseline: 1.0171x; 1.0076x over previous
import functools

import jax
import jax.numpy as jnp
from jax import lax
from jax.experimental import pallas as pl
from jax.experimental.pallas import tpu as pltpu
from jax.experimental.pallas import tpu_sc as plsc

D_MODEL = 128
NUM_CORES = 2
NUM_SUBCORES = 16
NUM_WORKERS = NUM_CORES * NUM_SUBCORES

CHUNK = 256
NBUF = 3
DEPTH = 3


@functools.lru_cache(maxsize=None)
def _make_gather(batch: int, seq: int):
    B = batch * seq
    assert B % NUM_WORKERS == 0
    b_per_w = B // NUM_WORKERS
    assert seq % b_per_w == 0
    w_per_row = seq // b_per_w
    assert b_per_w % CHUNK == 0
    n_chunks = b_per_w // CHUNK
    mesh = plsc.VectorSubcoreMesh(core_axis_name="c", subcore_axis_name="s")

    @functools.partial(
        pl.kernel,
        mesh=mesh,
        out_type=jax.ShapeDtypeStruct((B, D_MODEL), jnp.float32),
        scratch_types=[
            pltpu.VMEM((b_per_w,), jnp.int32),
            pltpu.VMEM((NBUF, CHUNK, D_MODEL), jnp.float32),
        ]
        + [pltpu.SemaphoreType.DMA] * (2 * NBUF + n_chunks),
    )
    def grab(idx_hbm, table_hbm, out_hbm, idx_v, rows_v, *sems):
        gsem, wsem = sems[:NBUF], sems[NBUF : 2 * NBUF]
        isem = sems[2 * NBUF :]
        wid = lax.axis_index("s") * NUM_CORES + lax.axis_index("c")
        base = wid * b_per_w
        row = wid // w_per_row
        col = (wid % w_per_row) * b_per_w

        idx_loads = [
            pltpu.async_copy(
                idx_hbm.at[row, pl.ds(col + c * CHUNK, CHUNK)],
                idx_v.at[pl.ds(c * CHUNK, CHUNK)],
                isem[c],
            )
            for c in range(n_chunks)
        ]

        gathers = [None] * n_chunks
        writes = [None] * n_chunks

        def start_write(d):
            gathers[d].wait()
            writes[d] = pltpu.async_copy(
                rows_v.at[d % NBUF],
                out_hbm.at[pl.ds(base + d * CHUNK, CHUNK)],
                wsem[d % NBUF],
            )

        for c in range(n_chunks):
            b = c % NBUF
            if c >= NBUF:
                writes[c - NBUF].wait()
            idx_loads[c].wait()
            gathers[c] = pltpu.async_copy(
                table_hbm.at[idx_v.at[pl.ds(c * CHUNK, CHUNK)]],
                rows_v.at[b],
                gsem[b],
            )
            if c - (DEPTH - 1) >= 0:
                start_write(c - (DEPTH - 1))
        for d in range(n_chunks - (DEPTH - 1), n_chunks):
            start_write(d)
        for d in range(max(0, n_chunks - NBUF), n_chunks):
            writes[d].wait()

    return grab


def kernel(x, table):
    batch, seq = x.shape
    out = _make_gather(batch, seq)(x.astype(jnp.int32), table)
    return out.reshape(batch, seq, D_MODEL)
